# baseline (speedup 1.0000x reference)
import functools

import jax
import jax.numpy as jnp
from jax import lax
from jax.experimental import pallas as pl
from jax.experimental.pallas import tpu as pltpu
from jax.experimental.pallas import tpu_sc as plsc

_NUM_USERS = 500000
_EMB = 64
_B = 16384
_NSC = 2
_H = _B // _NSC
_CHUNK = 512
_DEPTH = 8


def _body(users_hbm, items_hbm, neg_hbm, table_hbm,
          u_out, v_out, n_out,
          idx_s, *sems):
    c = lax.axis_index("c")
    base = c * _H

    for idx_hbm, dst_hbm, off in ((users_hbm, u_out, 0),
                                  (items_hbm, v_out, _NUM_USERS),
                                  (neg_hbm, n_out, _NUM_USERS)):
        for ch in range(_H // _CHUNK):
            pltpu.sync_copy(idx_hbm.at[pl.ds(base + ch * _CHUNK, _CHUNK)],
                            idx_s)

            def chunk_body(k0, dst_hbm=dst_hbm, off=off, ch=ch):
                for b in range(_DEPTH):
                    k = k0 + b

                    @pl.when(k0 > 0)
                    def _():
                        pltpu.make_async_copy(
                            table_hbm.at[pl.ds(0, 1)],
                            dst_hbm.at[pl.ds(0, 1)],
                            sems[b]).wait()

                    i = idx_s[k] + off
                    j = base + ch * _CHUNK + k
                    pltpu.async_copy(table_hbm.at[pl.ds(i, 1)],
                                     dst_hbm.at[pl.ds(j, 1)],
                                     sems[b])

            pl.loop(0, _CHUNK, step=_DEPTH)(chunk_body)
            for b in range(_DEPTH):
                pltpu.make_async_copy(
                    table_hbm.at[pl.ds(0, 1)],
                    dst_hbm.at[pl.ds(0, 1)],
                    sems[b]).wait()


_gather = functools.partial(
    pl.kernel,
    mesh=plsc.ScalarSubcoreMesh(axis_name="c", num_cores=_NSC),
    out_type=[jax.ShapeDtypeStruct((_B, _EMB), jnp.float32)] * 3,
    scratch_types=[pltpu.SMEM((_CHUNK,), jnp.int32)]
    + [pltpu.SemaphoreType.DMA] * _DEPTH,
)(_body)


def kernel(users, items, neg_items, U_and_V):
    u, v, n = _gather(users.astype(jnp.int32), items.astype(jnp.int32),
                      neg_items.astype(jnp.int32), U_and_V)
    return (u, v, n)


# speedup vs baseline: 3.0508x; 3.0508x over previous
import functools

import jax
import jax.numpy as jnp
from jax import lax
from jax.experimental import pallas as pl
from jax.experimental.pallas import tpu as pltpu
from jax.experimental.pallas import tpu_sc as plsc

_NUM_USERS = 500000
_EMB = 64
_B = 16384
_NSC = 2
_H = _B // _NSC
_CHUNK = 512
_DEPTH = 8


def _body(users_hbm, items_hbm, neg_hbm, table_hbm,
          u_out, v_out, n_out,
          idx_s, *sems):
    c = lax.axis_index("c")
    base = c * _H

    for idx_hbm, dst_hbm, off in ((users_hbm, u_out, 0),
                                  (items_hbm, v_out, _NUM_USERS),
                                  (neg_hbm, n_out, _NUM_USERS)):
        for ch in range(_H // _CHUNK):
            pltpu.sync_copy(idx_hbm.at[pl.ds(base + ch * _CHUNK, _CHUNK)],
                            idx_s)

            def issue_body(k, dst_hbm=dst_hbm, off=off, ch=ch):
                i = idx_s[k] + off
                j = base + ch * _CHUNK + k
                pltpu.async_copy(table_hbm.at[pl.ds(i, 1)],
                                 dst_hbm.at[pl.ds(j, 1)],
                                 sems[0])

            pl.loop(0, _CHUNK, unroll=4)(issue_body)

            def drain_body(k, dst_hbm=dst_hbm):
                pltpu.make_async_copy(
                    table_hbm.at[pl.ds(0, 1)],
                    dst_hbm.at[pl.ds(0, 1)],
                    sems[0]).wait()

            pl.loop(0, _CHUNK, unroll=4)(drain_body)


_gather = functools.partial(
    pl.kernel,
    mesh=plsc.ScalarSubcoreMesh(axis_name="c", num_cores=_NSC),
    out_type=[jax.ShapeDtypeStruct((_B, _EMB), jnp.float32)] * 3,
    scratch_types=[pltpu.SMEM((_CHUNK,), jnp.int32)]
    + [pltpu.SemaphoreType.DMA] * _DEPTH,
)(_body)


def kernel(users, items, neg_items, U_and_V):
    u, v, n = _gather(users.astype(jnp.int32), items.astype(jnp.int32),
                      neg_items.astype(jnp.int32), U_and_V)
    return (u, v, n)


# speedup vs baseline: 4.3204x; 1.4161x over previous
import functools

import jax
import jax.numpy as jnp
from jax import lax
from jax.experimental import pallas as pl
from jax.experimental.pallas import tpu as pltpu
from jax.experimental.pallas import tpu_sc as plsc
from jax._src.pallas import mpmd

_NUM_USERS = 500000
_EMB = 64
_B = 16384
_NC = 2
_NS = 16
_NW = _NC * _NS

_TEC_B = 10240
_TPW = _TEC_B // _NW
_SCS_B = _B - _TEC_B
_SPS = _SCS_B // _NC
_CHUNK = 512
_L = 16


def _tec_body(users_hbm, items_hbm, neg_hbm, table_hbm,
              u_out, v_out, n_out,
              idx_u, idx_i, idx_n, rows_u, rows_i, rows_n,
              sem_u, sem_i, sem_n, sem_o, idx_scs, sem_scs):
    wid = lax.axis_index("s") * _NC + lax.axis_index("c")
    base = wid * _TPW

    pltpu.sync_copy(users_hbm.at[pl.ds(base, _TPW)], idx_u)
    pltpu.sync_copy(items_hbm.at[pl.ds(base, _TPW)], idx_i)
    pltpu.sync_copy(neg_hbm.at[pl.ds(base, _TPW)], idx_n)
    for j in range(_TPW // _L):
        s = pl.ds(j * _L, _L)
        idx_i[s] = idx_i[s] + _NUM_USERS
        idx_n[s] = idx_n[s] + _NUM_USERS

    cu = pltpu.async_copy(table_hbm.at[idx_u], rows_u, sem_u)
    ci = pltpu.async_copy(table_hbm.at[idx_i], rows_i, sem_i)
    cn = pltpu.async_copy(table_hbm.at[idx_n], rows_n, sem_n)
    cu.wait()
    ou = pltpu.async_copy(rows_u, u_out.at[pl.ds(base, _TPW)], sem_o)
    ci.wait()
    oi = pltpu.async_copy(rows_i, v_out.at[pl.ds(base, _TPW)], sem_o)
    cn.wait()
    on = pltpu.async_copy(rows_n, n_out.at[pl.ds(base, _TPW)], sem_o)
    ou.wait()
    oi.wait()
    on.wait()


def _scs_body(users_hbm, items_hbm, neg_hbm, table_hbm,
              u_out, v_out, n_out,
              idx_u, idx_i, idx_n, rows_u, rows_i, rows_n,
              sem_u, sem_i, sem_n, sem_o, idx_scs, sem_scs):
    c = lax.axis_index("c")
    base = _TEC_B + c * _SPS

    for idx_hbm, dst_hbm, off in ((users_hbm, u_out, 0),
                                  (items_hbm, v_out, _NUM_USERS),
                                  (neg_hbm, n_out, _NUM_USERS)):
        for ch in range(_SPS // _CHUNK):
            pltpu.sync_copy(idx_hbm.at[pl.ds(base + ch * _CHUNK, _CHUNK)],
                            idx_scs)

            def issue_body(k, dst_hbm=dst_hbm, off=off, ch=ch):
                i = idx_scs[k] + off
                j = base + ch * _CHUNK + k
                pltpu.async_copy(table_hbm.at[pl.ds(i, 1)],
                                 dst_hbm.at[pl.ds(j, 1)],
                                 sem_scs)

            pl.loop(0, _CHUNK, unroll=4)(issue_body)

            def drain_body(k, dst_hbm=dst_hbm):
                pltpu.make_async_copy(
                    table_hbm.at[pl.ds(0, 1)],
                    dst_hbm.at[pl.ds(0, 1)],
                    sem_scs).wait()

            pl.loop(0, _CHUNK, unroll=4)(drain_body)


_scalar_mesh = plsc.ScalarSubcoreMesh(axis_name="c", num_cores=_NC)
_vector_mesh = plsc.VectorSubcoreMesh(core_axis_name="c", subcore_axis_name="s")

_gather = mpmd.mpmd_map(
    [(_scalar_mesh, _scs_body), (_vector_mesh, _tec_body)],
    out_types=[jax.ShapeDtypeStruct((_B, _EMB), jnp.float32)] * 3,
    scratch_types=[
        (pltpu.VMEM @ _vector_mesh)((_TPW,), jnp.int32),
        (pltpu.VMEM @ _vector_mesh)((_TPW,), jnp.int32),
        (pltpu.VMEM @ _vector_mesh)((_TPW,), jnp.int32),
        (pltpu.VMEM @ _vector_mesh)((_TPW, _EMB), jnp.float32),
        (pltpu.VMEM @ _vector_mesh)((_TPW, _EMB), jnp.float32),
        (pltpu.VMEM @ _vector_mesh)((_TPW, _EMB), jnp.float32),
        pltpu.SemaphoreType.DMA @ _vector_mesh,
        pltpu.SemaphoreType.DMA @ _vector_mesh,
        pltpu.SemaphoreType.DMA @ _vector_mesh,
        pltpu.SemaphoreType.DMA @ _vector_mesh,
        (pltpu.SMEM @ _scalar_mesh)((_CHUNK,), jnp.int32),
        pltpu.SemaphoreType.DMA @ _scalar_mesh,
    ],
    compiler_params=pltpu.CompilerParams(use_tc_tiling_on_sc=False),
)


def kernel(users, items, neg_items, U_and_V):
    u, v, n = _gather(users.astype(jnp.int32), items.astype(jnp.int32),
                      neg_items.astype(jnp.int32), U_and_V)
    return (u, v, n)


# speedup vs baseline: 4.3269x; 1.0015x over previous
import functools

import jax
import jax.numpy as jnp
from jax import lax
from jax.experimental import pallas as pl
from jax.experimental.pallas import tpu as pltpu
from jax.experimental.pallas import tpu_sc as plsc
from jax._src.pallas import mpmd

_NUM_USERS = 500000
_EMB = 64
_B = 16384
_NC = 2
_NS = 16
_NW = _NC * _NS

_TEC_B = 10240
_TPW = _TEC_B // _NW
_SCS_B = _B - _TEC_B
_SPS = _SCS_B // _NC
_CHUNK = 512
_L = 16


def _tec_body(users_hbm, items_hbm, neg_hbm, table_hbm,
              u_out, v_out, n_out,
              idx_u, idx_i, idx_n, rows_u, rows_i, rows_n,
              sem_u, sem_i, sem_n, sem_o, idx_scs, sem_scs):
    wid = lax.axis_index("s") * _NC + lax.axis_index("c")
    base = wid * _TPW

    pltpu.sync_copy(users_hbm.at[pl.ds(base, _TPW)], idx_u)
    pltpu.sync_copy(items_hbm.at[pl.ds(base, _TPW)], idx_i)
    pltpu.sync_copy(neg_hbm.at[pl.ds(base, _TPW)], idx_n)
    for j in range(_TPW // _L):
        s = pl.ds(j * _L, _L)
        idx_i[s] = idx_i[s] + _NUM_USERS
        idx_n[s] = idx_n[s] + _NUM_USERS

    cu = pltpu.async_copy(table_hbm.at[idx_u], rows_u, sem_u)
    ci = pltpu.async_copy(table_hbm.at[idx_i], rows_i, sem_i)
    cn = pltpu.async_copy(table_hbm.at[idx_n], rows_n, sem_n)
    cu.wait()
    ou = pltpu.async_copy(rows_u, u_out.at[pl.ds(base, _TPW)], sem_o)
    ci.wait()
    oi = pltpu.async_copy(rows_i, v_out.at[pl.ds(base, _TPW)], sem_o)
    cn.wait()
    on = pltpu.async_copy(rows_n, n_out.at[pl.ds(base, _TPW)], sem_o)
    ou.wait()
    oi.wait()
    on.wait()


def _scs_body(users_hbm, items_hbm, neg_hbm, table_hbm,
              u_out, v_out, n_out,
              idx_u, idx_i, idx_n, rows_u, rows_i, rows_n,
              sem_u, sem_i, sem_n, sem_o, idx_scs, sem_scs):
    c = lax.axis_index("c")
    base = _TEC_B + c * _SPS

    for idx_hbm, dst_hbm, off in ((users_hbm, u_out, 0),
                                  (items_hbm, v_out, _NUM_USERS),
                                  (neg_hbm, n_out, _NUM_USERS)):
        for ch in range(_SPS // _CHUNK):
            pltpu.sync_copy(idx_hbm.at[pl.ds(base + ch * _CHUNK, _CHUNK)],
                            idx_scs)

            def issue_body(k, dst_hbm=dst_hbm, off=off, ch=ch):
                i = idx_scs[k] + off
                j = base + ch * _CHUNK + k
                pltpu.async_copy(table_hbm.at[pl.ds(i, 1)],
                                 dst_hbm.at[pl.ds(j, 1)],
                                 sem_scs)

            pl.loop(0, _CHUNK, unroll=4)(issue_body)

            def drain_body(k, dst_hbm=dst_hbm):
                pltpu.make_async_copy(
                    table_hbm.at[pl.ds(0, 1)],
                    dst_hbm.at[pl.ds(0, 1)],
                    sem_scs).wait()

            pl.loop(0, _CHUNK, unroll=4)(drain_body)


_scalar_mesh = plsc.ScalarSubcoreMesh(axis_name="c", num_cores=_NC)
_vector_mesh = plsc.VectorSubcoreMesh(core_axis_name="c", subcore_axis_name="s")

_gather = mpmd.mpmd_map(
    [(_vector_mesh, _tec_body), (_scalar_mesh, _scs_body)],
    out_types=[jax.ShapeDtypeStruct((_B, _EMB), jnp.float32)] * 3,
    scratch_types=[
        (pltpu.VMEM @ _vector_mesh)((_TPW,), jnp.int32),
        (pltpu.VMEM @ _vector_mesh)((_TPW,), jnp.int32),
        (pltpu.VMEM @ _vector_mesh)((_TPW,), jnp.int32),
        (pltpu.VMEM @ _vector_mesh)((_TPW, _EMB), jnp.float32),
        (pltpu.VMEM @ _vector_mesh)((_TPW, _EMB), jnp.float32),
        (pltpu.VMEM @ _vector_mesh)((_TPW, _EMB), jnp.float32),
        pltpu.SemaphoreType.DMA @ _vector_mesh,
        pltpu.SemaphoreType.DMA @ _vector_mesh,
        pltpu.SemaphoreType.DMA @ _vector_mesh,
        pltpu.SemaphoreType.DMA @ _vector_mesh,
        (pltpu.SMEM @ _scalar_mesh)((_CHUNK,), jnp.int32),
        pltpu.SemaphoreType.DMA @ _scalar_mesh,
    ],
    compiler_params=pltpu.CompilerParams(use_tc_tiling_on_sc=False),
)


def kernel(users, items, neg_items, U_and_V):
    u, v, n = _gather(users.astype(jnp.int32), items.astype(jnp.int32),
                      neg_items.astype(jnp.int32), U_and_V)
    return (u, v, n)


# speedup vs baseline: 4.5575x; 1.0533x over previous
import functools

import jax
import jax.numpy as jnp
from jax import lax
from jax.experimental import pallas as pl
from jax.experimental.pallas import tpu as pltpu
from jax.experimental.pallas import tpu_sc as plsc

_NUM_USERS = 500000
_EMB = 64
_B = 16384
_NC = 2
_NS = 16
_NW = _NC * _NS
_BPW = _B // _NW


def _body(users_hbm, items_hbm, neg_hbm, table_hbm,
          u_out, v_out, n_out,
          idx_v, idx_sh, idx_s, rows_sh, rows_v, sem_g, sem_o):
    sid = lax.axis_index("s")
    wid = sid * _NC + lax.axis_index("c")
    base = wid * _BPW
    slot0 = sid * _BPW

    for oi, (idx_hbm, dst_hbm, off) in enumerate(
            ((users_hbm, u_out, 0),
             (items_hbm, v_out, _NUM_USERS),
             (neg_hbm, n_out, _NUM_USERS))):
        pltpu.sync_copy(idx_hbm.at[pl.ds(base, _BPW)], idx_v)
        pltpu.sync_copy(idx_v, idx_sh.at[sid])
        pltpu.sync_copy(idx_sh.at[sid], idx_s)

        def issue_body(k, off=off):
            i = idx_s[k] + off
            pltpu.async_copy(table_hbm.at[pl.ds(i, 1)],
                             rows_sh.at[pl.ds(slot0 + k, 1)], sem_g)

        pl.loop(0, _BPW, unroll=8)(issue_body)

        def drain_body(k):
            pltpu.make_async_copy(table_hbm.at[pl.ds(0, 1)],
                                  rows_sh.at[pl.ds(0, 1)], sem_g).wait()

        pl.loop(0, _BPW, unroll=8)(drain_body)

        pltpu.sync_copy(rows_sh.at[pl.ds(slot0, _BPW)], rows_v)
        ocp = pltpu.async_copy(rows_v, dst_hbm.at[pl.ds(base, _BPW)], sem_o)
        ocp.wait()


_gather = functools.partial(
    pl.kernel,
    mesh=plsc.VectorSubcoreMesh(core_axis_name="c", subcore_axis_name="s"),
    compiler_params=pltpu.CompilerParams(use_tc_tiling_on_sc=False),
    out_type=[jax.ShapeDtypeStruct((_B, _EMB), jnp.float32)] * 3,
    scratch_types=[
        pltpu.VMEM((_BPW,), jnp.int32),
        pltpu.VMEM_SHARED((_NS, _BPW), jnp.int32),
        pltpu.SMEM((_BPW,), jnp.int32),
        pltpu.VMEM_SHARED((_NS * _BPW, _EMB), jnp.float32),
        pltpu.VMEM((_BPW, _EMB), jnp.float32),
        pltpu.SemaphoreType.DMA,
        pltpu.SemaphoreType.DMA,
    ],
)(_body)


def kernel(users, items, neg_items, U_and_V):
    u, v, n = _gather(users.astype(jnp.int32), items.astype(jnp.int32),
                      neg_items.astype(jnp.int32), U_and_V)
    return (u, v, n)


# speedup vs baseline: 5.2324x; 1.1481x over previous
import functools

import jax
import jax.numpy as jnp
from jax import lax
from jax.experimental import pallas as pl
from jax.experimental.pallas import tpu as pltpu
from jax.experimental.pallas import tpu_sc as plsc

_NUM_USERS = 500000
_EMB = 64
_B = 16384
_NC = 2
_NS = 16
_NW = _NC * _NS
_BPW = _B // _NW
_L = 16

_NCHUNK = 8
_CS = _BPW // _NCHUNK


def _body(users_hbm, items_hbm, neg_hbm, table_hbm,
          u_out, v_out, n_out,
          idx_u, idx_i, idx_n, rows_u, rows_i, rows_n,
          sem_u, sem_i, sem_n, sem_o):
    wid = lax.axis_index("s") * _NC + lax.axis_index("c")
    base = wid * _BPW

    pltpu.sync_copy(users_hbm.at[pl.ds(base, _BPW)], idx_u)
    pltpu.sync_copy(items_hbm.at[pl.ds(base, _BPW)], idx_i)
    pltpu.sync_copy(neg_hbm.at[pl.ds(base, _BPW)], idx_n)
    for j in range(_BPW // _L):
        s = pl.ds(j * _L, _L)
        idx_i[s] = idx_i[s] + _NUM_USERS
        idx_n[s] = idx_n[s] + _NUM_USERS

    cps = []
    for c in range(_NCHUNK):
        s = pl.ds(c * _CS, _CS)
        cps.append(pltpu.async_copy(table_hbm.at[idx_u.at[s]], rows_u.at[s], sem_u))
        cps.append(pltpu.async_copy(table_hbm.at[idx_i.at[s]], rows_i.at[s], sem_i))
        cps.append(pltpu.async_copy(table_hbm.at[idx_n.at[s]], rows_n.at[s], sem_n))
    for cp in cps:
        cp.wait()
    ou = pltpu.async_copy(rows_u, u_out.at[pl.ds(base, _BPW)], sem_o)
    oi = pltpu.async_copy(rows_i, v_out.at[pl.ds(base, _BPW)], sem_o)
    on = pltpu.async_copy(rows_n, n_out.at[pl.ds(base, _BPW)], sem_o)
    ou.wait()
    oi.wait()
    on.wait()


_gather = functools.partial(
    pl.kernel,
    mesh=plsc.VectorSubcoreMesh(core_axis_name="c", subcore_axis_name="s"),
    compiler_params=pltpu.CompilerParams(use_tc_tiling_on_sc=False),
    out_type=[jax.ShapeDtypeStruct((_B, _EMB), jnp.float32)] * 3,
    scratch_types=[
        pltpu.VMEM((_BPW,), jnp.int32),
        pltpu.VMEM((_BPW,), jnp.int32),
        pltpu.VMEM((_BPW,), jnp.int32),
        pltpu.VMEM((_BPW, _EMB), jnp.float32),
        pltpu.VMEM((_BPW, _EMB), jnp.float32),
        pltpu.VMEM((_BPW, _EMB), jnp.float32),
        pltpu.SemaphoreType.DMA,
        pltpu.SemaphoreType.DMA,
        pltpu.SemaphoreType.DMA,
        pltpu.SemaphoreType.DMA,
    ],
)(_body)


def kernel(users, items, neg_items, U_and_V):
    u, v, n = _gather(users.astype(jnp.int32), items.astype(jnp.int32),
                      neg_items.astype(jnp.int32), U_and_V)
    return (u, v, n)
